# baseline (speedup 1.0000x reference)
import functools

import numpy as np
import jax
import jax.numpy as jnp
from jax import lax
from jax.experimental import pallas as pl
from jax.experimental.pallas import tpu as pltpu
from jax.experimental.pallas import tpu_sc as plsc

_VOCAB = 1000000
_MAXLEN = 200
_EMBED = 32
_BATCH = 4096

_NC = 2
_NS = 16
_NW = _NC * _NS

_N = _BATCH * _MAXLEN
_PER_W = _N // _NW
_SEQS_PER_CHUNK = 4
_CHUNK = _SEQS_PER_CHUNK * _MAXLEN
_NCHUNK = _PER_W // _CHUNK
_NBUF = 4
_NOBUF = 2
_NGRP = _NCHUNK // _NBUF

_RPK_ROWS = _VOCAB // 4
_RPK_BLK = 1000
_RPK_GRID = _RPK_ROWS // _RPK_BLK


def _repack_body(x_ref, o_ref):
    u = lax.bitcast_convert_type(x_ref[...], jnp.int32)
    lsb = lax.bitwise_and(lax.shift_right_logical(u, 16), 1)
    bf = lax.shift_right_logical(u + 0x7FFF + lsb, 16)
    lo = bf[:, 0:16]
    hi = bf[:, 16:32]
    o_ref[...] = lax.bitwise_or(lo, lax.shift_left(hi, 16))


def _repack(token_table):
    out = pl.pallas_call(
        _repack_body,
        grid=(_RPK_GRID,),
        in_specs=[pl.BlockSpec((4 * _RPK_BLK, _EMBED), lambda i: (i, 0))],
        out_specs=pl.BlockSpec((4 * _RPK_BLK, _EMBED // 2), lambda i: (i, 0)),
        out_shape=jax.ShapeDtypeStruct((_VOCAB, _EMBED // 2), jnp.int32),
    )(token_table)
    return out


def _tpe(xf, tok_packed, pos_table):
    mesh = plsc.VectorSubcoreMesh(core_axis_name="c", subcore_axis_name="s")

    scratch = (
        [pltpu.VMEM((_CHUNK,), jnp.int32) for _ in range(_NBUF)]
        + [pltpu.VMEM((_CHUNK, _EMBED // 2), jnp.int32) for _ in range(_NBUF)]
        + [pltpu.VMEM((_CHUNK, _EMBED), jnp.float32) for _ in range(_NOBUF)]
        + [pltpu.VMEM((_MAXLEN, _EMBED), jnp.float32)]
        + [pltpu.SemaphoreType.DMA for _ in range(2 * _NBUF + _NOBUF)]
    )

    @functools.partial(
        pl.kernel,
        out_type=jax.ShapeDtypeStruct((_N, _EMBED), jnp.float32),
        mesh=mesh,
        compiler_params=pltpu.CompilerParams(use_tc_tiling_on_sc=False),
        scratch_types=scratch,
    )
    def k(x_hbm, tok_hbm, pos_hbm, out_hbm, *sc):
        ibuf = sc[0:_NBUF]
        gbuf = sc[_NBUF:2 * _NBUF]
        obuf = sc[2 * _NBUF:2 * _NBUF + _NOBUF]
        pos_v = sc[2 * _NBUF + _NOBUF]
        base_sem = 2 * _NBUF + _NOBUF + 1
        isem = sc[base_sem:base_sem + _NBUF]
        gsem = sc[base_sem + _NBUF:base_sem + 2 * _NBUF]
        osem = sc[base_sem + 2 * _NBUF:base_sem + 2 * _NBUF + _NOBUF]

        wid = lax.axis_index("s") * _NC + lax.axis_index("c")
        base0 = wid * _PER_W
        pltpu.sync_copy(pos_hbm, pos_v)

        def fire_idx(c, b):
            pltpu.async_copy(
                x_hbm.at[pl.ds(base0 + c * _CHUNK, _CHUNK)], ibuf[b], isem[b]
            )

        def wait_idx(b):
            pltpu.make_async_copy(
                x_hbm.at[pl.ds(0, _CHUNK)], ibuf[b], isem[b]
            ).wait()

        def fire_gather(c, b):
            pltpu.async_copy(tok_hbm.at[ibuf[b]], gbuf[b], gsem[b])

        def wait_gather(b):
            pltpu.make_async_copy(
                tok_hbm.at[pl.ds(0, _CHUNK)], gbuf[b], gsem[b]
            ).wait()

        def unpack_add(b, ob):
            hi_mask = jnp.full((16,), -65536, dtype=jnp.int32)
            sh16 = jnp.full((16,), 16, dtype=jnp.int32)

            def m_body(m, carry):
                p0 = pos_v[m, 0:16]
                p1 = pos_v[m, 16:32]
                for s in range(_SEQS_PER_CHUNK):
                    r = s * _MAXLEN + m
                    w = gbuf[b][r, 0:16]
                    v0 = lax.bitcast_convert_type(lax.shift_left(w, sh16), jnp.float32)
                    v1 = lax.bitcast_convert_type(lax.bitwise_and(w, hi_mask), jnp.float32)
                    obuf[ob][r, 0:16] = v0 + p0
                    obuf[ob][r, 16:32] = v1 + p1
                return carry

            lax.fori_loop(0, _MAXLEN, m_body, 0, unroll=4)

        def fire_out(c, ob):
            pltpu.async_copy(
                obuf[ob], out_hbm.at[pl.ds(base0 + c * _CHUNK, _CHUNK)], osem[ob]
            )

        def wait_out(ob):
            pltpu.make_async_copy(
                obuf[ob], out_hbm.at[pl.ds(0, _CHUNK)], osem[ob]
            ).wait()

        def step(c, b, first, last):
            b1 = (b + 1) % _NBUF
            ob = b % _NOBUF
            wait_gather(b)
            if not last:
                wait_idx(b1)
                fire_gather(c + 1, b1)
            if not (first and b < _NOBUF):
                wait_out(ob)
            unpack_add(b, ob)
            fire_out(c, ob)
            if not last:
                fire_idx(c + _NBUF, b)

        for b in range(_NBUF):
            fire_idx(b, b)
        wait_idx(0)
        fire_gather(0, 0)

        for b in range(_NBUF):
            step(b, b, first=True, last=False)

        def group(cc, carry):
            c0 = cc * _NBUF
            for b in range(_NBUF):
                step(c0 + b, b, first=False, last=False)
            return carry

        lax.fori_loop(1, _NGRP - 1, group, 0)

        c0 = (_NGRP - 1) * _NBUF
        for b in range(_NBUF):
            b1 = (b + 1) % _NBUF
            ob = b % _NOBUF
            wait_gather(b)
            if b < _NBUF - 1:
                wait_idx(b1)
                fire_gather(c0 + b + 1, b1)
            wait_out(ob)
            unpack_add(b, ob)
            fire_out(c0 + b, ob)
        for ob in range(_NOBUF):
            wait_out(ob)

    return k(xf, tok_packed, pos_table)


def kernel(x, token_table, pos_table):
    xf = x.reshape(-1).astype(jnp.int32)
    tok_packed = _repack(token_table)
    out = _tpe(xf, tok_packed, pos_table)
    return out.reshape(x.shape[0], x.shape[1], _EMBED)


# speedup vs baseline: 1.5844x; 1.5844x over previous
import functools

import jax
import jax.numpy as jnp
from jax import lax
from jax.experimental import pallas as pl
from jax.experimental.pallas import tpu as pltpu
from jax.experimental.pallas import tpu_sc as plsc

_VOCAB = 1000000
_MAXLEN = 200
_EMBED = 32
_BATCH = 4096

_NC = 2
_NS = 16
_NW = _NC * _NS

_N = _BATCH * _MAXLEN
_PER_W = _N // _NW
_SEQS_PER_CHUNK = 4
_CHUNK = _SEQS_PER_CHUNK * _MAXLEN
_NCHUNK = _PER_W // _CHUNK
_NBUF = 4
_NGRP = _NCHUNK // _NBUF


def _tpe(xf, token_table, pos_table):
    mesh = plsc.VectorSubcoreMesh(core_axis_name="c", subcore_axis_name="s")

    scratch = (
        [pltpu.VMEM((_CHUNK,), jnp.int32) for _ in range(_NBUF)]
        + [pltpu.VMEM((_CHUNK, _EMBED), jnp.float32) for _ in range(_NBUF)]
        + [pltpu.VMEM((_MAXLEN, _EMBED), jnp.float32)]
        + [pltpu.SemaphoreType.DMA for _ in range(3 * _NBUF)]
    )

    @functools.partial(
        pl.kernel,
        out_type=jax.ShapeDtypeStruct((_N, _EMBED), jnp.float32),
        mesh=mesh,
        compiler_params=pltpu.CompilerParams(use_tc_tiling_on_sc=False),
        scratch_types=scratch,
    )
    def k(x_hbm, tok_hbm, pos_hbm, out_hbm, *sc):
        ibuf = sc[0:_NBUF]
        gbuf = sc[_NBUF:2 * _NBUF]
        pos_v = sc[2 * _NBUF]
        isem = sc[2 * _NBUF + 1:3 * _NBUF + 1]
        gsem = sc[3 * _NBUF + 1:4 * _NBUF + 1]
        osem = sc[4 * _NBUF + 1:5 * _NBUF + 1]

        wid = lax.axis_index("s") * _NC + lax.axis_index("c")
        base0 = wid * _PER_W
        pltpu.sync_copy(pos_hbm, pos_v)

        def fire_idx(c, b):
            pltpu.async_copy(
                x_hbm.at[pl.ds(base0 + c * _CHUNK, _CHUNK)], ibuf[b], isem[b]
            )

        def wait_idx(b):
            pltpu.make_async_copy(
                x_hbm.at[pl.ds(0, _CHUNK)], ibuf[b], isem[b]
            ).wait()

        def fire_gather(c, b):
            pltpu.async_copy(tok_hbm.at[ibuf[b]], gbuf[b], gsem[b])

        def wait_gather(b):
            pltpu.make_async_copy(
                out_hbm.at[pl.ds(0, _CHUNK)], gbuf[b], gsem[b]
            ).wait()

        def add_pos(b):
            def m_body(m, carry):
                p0 = pos_v[m, 0:16]
                p1 = pos_v[m, 16:32]
                for s in range(_SEQS_PER_CHUNK):
                    r = s * _MAXLEN + m
                    plsc.addupdate(gbuf[b].at[r, pl.ds(0, 16)], p0)
                    plsc.addupdate(gbuf[b].at[r, pl.ds(16, 16)], p1)
                return carry

            lax.fori_loop(0, _MAXLEN, m_body, 0, unroll=4)

        def fire_out(c, b):
            pltpu.async_copy(
                gbuf[b], out_hbm.at[pl.ds(base0 + c * _CHUNK, _CHUNK)], osem[b]
            )

        def wait_out(b):
            pltpu.make_async_copy(
                gbuf[b], out_hbm.at[pl.ds(0, _CHUNK)], osem[b]
            ).wait()

        def step(c, b, first, last):
            b1 = (b + 1) % _NBUF
            wait_gather(b)
            if not last:
                wait_idx(b1)
                if not (first and b < _NBUF - 1):
                    wait_out(b1)
                fire_gather(c + 1, b1)
            add_pos(b)
            fire_out(c, b)
            if not last:
                fire_idx(c + _NBUF, b)

        for b in range(_NBUF):
            fire_idx(b, b)
        wait_idx(0)
        fire_gather(0, 0)

        for b in range(_NBUF):
            step(b, b, first=True, last=False)

        def group(cc, carry):
            c0 = cc * _NBUF
            for b in range(_NBUF):
                step(c0 + b, b, first=False, last=False)
            return carry

        lax.fori_loop(1, _NGRP - 1, group, 0)

        c0 = (_NGRP - 1) * _NBUF
        for b in range(_NBUF):
            b1 = (b + 1) % _NBUF
            wait_gather(b)
            if b < _NBUF - 1:
                wait_idx(b1)
                wait_out(b1)
                fire_gather(c0 + b + 1, b1)
            add_pos(b)
            fire_out(c0 + b, b)
        for b in range(_NBUF):
            wait_out(b)

    return k(xf, token_table, pos_table)


def kernel(x, token_table, pos_table):
    xf = x.reshape(-1).astype(jnp.int32)
    out = _tpe(xf, token_table, pos_table)
    return out.reshape(x.shape[0], x.shape[1], _EMBED)
